# baseline (speedup 1.0000x reference)
import functools

import jax
import jax.numpy as jnp
from jax.experimental import pallas as pl
from jax.experimental.pallas import tpu as pltpu

_B, _S, _H = 8, 1024, 512
_N = _B * _S
_K = 10
_NL = 4
_EPS = 1e-5
_ROWS = 1024


def _adj_kernel(car_ref, cac_ref, adj_ref):
    rblk = pl.program_id(1)
    xr = car_ref[0]
    xc = cac_ref[0]
    d0 = xr[:, 0:1] - xc[0:1, :]
    d1 = xr[:, 1:2] - xc[1:2, :]
    d2 = xr[:, 2:3] - xc[2:3, :]
    dist = jnp.sqrt(d0 * d0 + d1 * d1 + d2 * d2)
    col = jax.lax.broadcasted_iota(jnp.int32, (_ROWS, _S), 1)
    row = rblk * _ROWS + jax.lax.broadcasted_iota(jnp.int32, (_ROWS, 1), 0)

    def body(_, neg):
        m = jnp.max(neg, axis=1, keepdims=True)
        jstar = jnp.min(jnp.where(neg == m, col, _S), axis=1, keepdims=True)
        return jnp.where(col == jstar, -jnp.inf, neg)

    neg0 = jnp.where(col == row, -jnp.inf, -dist)
    neg = jax.lax.fori_loop(0, _K, body, neg0)
    picked = (neg == -jnp.inf) & (col != row)
    adj_ref[0] = picked.astype(jnp.bfloat16)


def _mm_body(h, a, w0, w1, w2, ws, bias, i, out_s, s1_s, s2_s, din):
    z = jnp.zeros((3, din), jnp.float32)
    hp = jnp.concatenate([z, h, z], axis=0)
    u0 = hp[4:4 + _S] + hp[5:5 + _S] + hp[6:6 + _S]
    u1 = hp[2:2 + _S] + hp[1:1 + _S] + hp[0:_S]
    hb = h.astype(jnp.bfloat16)
    u2 = jax.lax.dot_general(a, hb, (((0,), (0,)), ((), ())),
                             preferred_element_type=jnp.float32)
    acc = jnp.dot(u0.astype(jnp.bfloat16), w0,
                  preferred_element_type=jnp.float32)
    acc = acc + jnp.dot(u1.astype(jnp.bfloat16), w1,
                        preferred_element_type=jnp.float32)
    acc = acc + jnp.dot(u2.astype(jnp.bfloat16), w2,
                        preferred_element_type=jnp.float32)
    acc = acc + jnp.dot(hb, ws, preferred_element_type=jnp.float32)
    acc = acc + bias
    out_s[pl.ds(i * _S, _S), :] = acc
    s1_s[pl.ds(i, 1), :] = jnp.sum(acc, axis=0, keepdims=True)
    s2_s[pl.ds(i, 1), :] = jnp.sum(acc * acc, axis=0, keepdims=True)


def _bn1_body(g1, b1, g2, b2, out_s, h_s, s1_s, s2_s, st2_s, shortcut):
    inv_n = 1.0 / _N
    m1 = jnp.sum(s1_s[...], axis=0, keepdims=True) * inv_n
    v1 = jnp.sum(s2_s[...], axis=0, keepdims=True) * inv_n - m1 * m1
    sc1 = g1 / jnp.sqrt(v1 + _EPS)
    sh1 = b1 - m1 * sc1
    s1 = jnp.zeros((1, _H), jnp.float32)
    s2 = jnp.zeros((1, _H), jnp.float32)
    for c in range(_B):
        x = out_s[pl.ds(c * _S, _S), :]
        y = jnp.maximum(x * sc1 + sh1, 0.0)
        if shortcut:
            y = y + h_s[pl.ds(c * _S, _S), :]
        out_s[pl.ds(c * _S, _S), :] = y
        s1 = s1 + jnp.sum(y, axis=0, keepdims=True)
        s2 = s2 + jnp.sum(y * y, axis=0, keepdims=True)
    m2 = s1 * inv_n
    v2 = s2 * inv_n - m2 * m2
    sc2 = g2 / jnp.sqrt(v2 + _EPS)
    st2_s[0:1, :] = sc2
    st2_s[1:2, :] = b2 - m2 * sc2


def _layer0_kernel(h_ref, a_ref, w0_ref, w1_ref, w2_ref, ws_ref, bias_ref,
                   g1_ref, b1_ref, g2_ref, b2_ref, out_ref,
                   out_s, s1_s, s2_s, st2_s):
    i = pl.program_id(0)

    @pl.when(i < _B)
    def _mm():
        _mm_body(h_ref[...], a_ref[0], w0_ref[...], w1_ref[...], w2_ref[...],
                 ws_ref[...], bias_ref[...], i, out_s, s1_s, s2_s, din=3)

    @pl.when(i == _B)
    def _bn1():
        _bn1_body(g1_ref[...], b1_ref[...], g2_ref[...], b2_ref[...],
                  out_s, None, s1_s, s2_s, st2_s, shortcut=False)

    @pl.when(i > _B)
    def _bn2():
        b = i - _B - 1
        y = out_s[pl.ds(b * _S, _S), :]
        out_ref[...] = y * st2_s[0:1, :] + st2_s[1:2, :]


def _layers123_kernel(h_ref, a_ref, w0_ref, w1_ref, w2_ref, ws_ref, bias_ref,
                      g1_ref, b1_ref, g2_ref, b2_ref, out_ref,
                      out_s, h_s, s1_s, s2_s, st2_s):
    l = pl.program_id(0)
    i = pl.program_id(1)

    @pl.when((i < _B) & (l == 0))
    def _mm_first():
        h = h_ref[...]
        h_s[pl.ds(i * _S, _S), :] = h
        _mm_body(h, a_ref[0], w0_ref[0], w1_ref[0], w2_ref[0], ws_ref[0],
                 bias_ref[0], i, out_s, s1_s, s2_s, din=_H)

    @pl.when((i < _B) & (l > 0))
    def _mm_rest():
        h = h_s[pl.ds(i * _S, _S), :]
        _mm_body(h, a_ref[0], w0_ref[0], w1_ref[0], w2_ref[0], ws_ref[0],
                 bias_ref[0], i, out_s, s1_s, s2_s, din=_H)

    @pl.when(i == _B)
    def _bn1():
        _bn1_body(g1_ref[0], b1_ref[0], g2_ref[0], b2_ref[0],
                  out_s, h_s, s1_s, s2_s, st2_s, shortcut=True)

    @pl.when((i > _B) & (l < 2))
    def _bn2_inner():
        b = i - _B - 1
        y = out_s[pl.ds(b * _S, _S), :]
        h_s[pl.ds(b * _S, _S), :] = y * st2_s[0:1, :] + st2_s[1:2, :]

    @pl.when((i > _B) & (l == 2))
    def _bn2_last():
        b = i - _B - 1
        y = out_s[pl.ds(b * _S, _S), :]
        out_ref[...] = y * st2_s[0:1, :] + st2_s[1:2, :]


def kernel(n_coords, ca_coords, c_coords, Wl, bl, Ws, bs, g1, b1, g2, b2):
    ca = ca_coords.astype(jnp.float32)
    ca_cols = ca.transpose(0, 2, 1)
    adj = pl.pallas_call(
        _adj_kernel,
        grid=(_B, _S // _ROWS),
        in_specs=[pl.BlockSpec((1, _ROWS, 3), lambda b, r: (b, r, 0)),
                  pl.BlockSpec((1, 3, _S), lambda b, r: (b, 0, 0))],
        out_specs=pl.BlockSpec((1, _ROWS, _S), lambda b, r: (b, r, 0)),
        out_shape=jax.ShapeDtypeStruct((_B, _S, _S), jnp.bfloat16),
    )(ca, ca_cols)

    h0 = ca.reshape(_N, 3)
    w = Wl[0]
    wb0 = [x.astype(jnp.bfloat16) for x in (w[:3], w[3:6], w[6:9], Ws[0])]
    bias0 = (bl[0] + bs[0]).reshape(1, _H)
    wspec0 = pl.BlockSpec((3, _H), lambda i: (0, 0))
    vspec0 = pl.BlockSpec((1, _H), lambda i: (0, 0))
    h1 = pl.pallas_call(
        _layer0_kernel,
        grid=(2 * _B + 1,),
        in_specs=[pl.BlockSpec((_S, 3), lambda i: (jnp.minimum(i, _B - 1), 0)),
                  pl.BlockSpec((1, _S, _S),
                               lambda i: (jnp.minimum(i, _B - 1), 0, 0)),
                  wspec0, wspec0, wspec0, wspec0,
                  vspec0, vspec0, vspec0, vspec0, vspec0],
        out_specs=pl.BlockSpec(
            (_S, _H), lambda i: (jnp.maximum(i - _B - 1, 0), 0)),
        out_shape=jax.ShapeDtypeStruct((_N, _H), jnp.float32),
        scratch_shapes=[pltpu.VMEM((_N, _H), jnp.float32),
                        pltpu.VMEM((_B, _H), jnp.float32),
                        pltpu.VMEM((_B, _H), jnp.float32),
                        pltpu.VMEM((2, _H), jnp.float32)],
    )(h0, adj, *wb0, bias0, g1[0].reshape(1, _H), b1[0].reshape(1, _H),
      g2[0].reshape(1, _H), b2[0].reshape(1, _H))

    w0s = jnp.stack([Wl[l][:_H] for l in range(1, _NL)]).astype(jnp.bfloat16)
    w1s = jnp.stack([Wl[l][_H:2 * _H]
                     for l in range(1, _NL)]).astype(jnp.bfloat16)
    w2s = jnp.stack([Wl[l][2 * _H:3 * _H]
                     for l in range(1, _NL)]).astype(jnp.bfloat16)
    wss = jnp.stack([Ws[l] for l in range(1, _NL)]).astype(jnp.bfloat16)
    biases = jnp.stack([(bl[l] + bs[l]).reshape(1, _H)
                        for l in range(1, _NL)])
    g1s = jnp.stack([g1[l].reshape(1, _H) for l in range(1, _NL)])
    b1s = jnp.stack([b1[l].reshape(1, _H) for l in range(1, _NL)])
    g2s = jnp.stack([g2[l].reshape(1, _H) for l in range(1, _NL)])
    b2s = jnp.stack([b2[l].reshape(1, _H) for l in range(1, _NL)])
    wspec = pl.BlockSpec((1, _H, _H), lambda l, i: (l, 0, 0))
    vspec = pl.BlockSpec((1, 1, _H), lambda l, i: (l, 0, 0))
    h = pl.pallas_call(
        _layers123_kernel,
        grid=(_NL - 1, 2 * _B + 1),
        in_specs=[pl.BlockSpec((_S, _H),
                               lambda l, i: (jnp.minimum(i, _B - 1), 0)),
                  pl.BlockSpec((1, _S, _S),
                               lambda l, i: (jnp.minimum(i, _B - 1), 0, 0)),
                  wspec, wspec, wspec, wspec,
                  vspec, vspec, vspec, vspec, vspec],
        out_specs=pl.BlockSpec(
            (_S, _H), lambda l, i: (jnp.maximum(i - _B - 1, 0), 0)),
        out_shape=jax.ShapeDtypeStruct((_N, _H), jnp.float32),
        scratch_shapes=[pltpu.VMEM((_N, _H), jnp.float32),
                        pltpu.VMEM((_N, _H), jnp.float32),
                        pltpu.VMEM((_B, _H), jnp.float32),
                        pltpu.VMEM((_B, _H), jnp.float32),
                        pltpu.VMEM((2, _H), jnp.float32)],
    )(h1, adj, w0s, w1s, w2s, wss, biases, g1s, b1s, g2s, b2s)
    return h.reshape(_B, _S, _H)


# speedup vs baseline: 1.0099x; 1.0099x over previous
import functools

import jax
import jax.numpy as jnp
from jax.experimental import pallas as pl
from jax.experimental.pallas import tpu as pltpu

_B, _S, _H = 8, 1024, 512
_N = _B * _S
_K = 10
_NL = 4
_EPS = 1e-5
_ROWS = 1024


def _adj_kernel(car_ref, cac_ref, adj_ref):
    rblk = pl.program_id(1)
    xr = car_ref[0]
    xc = cac_ref[0]
    d0 = xr[:, 0:1] - xc[0:1, :]
    d1 = xr[:, 1:2] - xc[1:2, :]
    d2 = xr[:, 2:3] - xc[2:3, :]
    dist = jnp.sqrt(d0 * d0 + d1 * d1 + d2 * d2)
    col = jax.lax.broadcasted_iota(jnp.int32, (_ROWS, _S), 1)
    row = rblk * _ROWS + jax.lax.broadcasted_iota(jnp.int32, (_ROWS, 1), 0)

    def body(_, neg):
        m = jnp.max(neg, axis=1, keepdims=True)
        jstar = jnp.min(jnp.where(neg == m, col, _S), axis=1, keepdims=True)
        return jnp.where(col == jstar, -jnp.inf, neg)

    neg0 = jnp.where(col == row, -jnp.inf, -dist)
    neg = jax.lax.fori_loop(0, _K, body, neg0)
    picked = (neg == -jnp.inf) & (col != row)
    adj_ref[0] = picked.astype(jnp.bfloat16)


def _layer_kernel(h_ref, a_ref, w0_ref, w1_ref, w2_ref, ws_ref, bias_ref,
                  g1_ref, b1_ref, g2_ref, b2_ref, out_ref,
                  out_s, h_s, s1_s, s2_s, st2_s, *, din, shortcut):
    i = pl.program_id(0)

    @pl.when(i < _B)
    def _mm():
        h = h_ref[...]
        a = a_ref[0]
        z = jnp.zeros((3, din), jnp.float32)
        hp = jnp.concatenate([z, h, z], axis=0)
        u0 = hp[4:4 + _S] + hp[5:5 + _S] + hp[6:6 + _S]
        u1 = hp[2:2 + _S] + hp[1:1 + _S] + hp[0:_S]
        hb = h.astype(jnp.bfloat16)
        u2 = jax.lax.dot_general(a, hb, (((0,), (0,)), ((), ())),
                                 preferred_element_type=jnp.float32)
        acc = jnp.dot(u0.astype(jnp.bfloat16), w0_ref[...],
                      preferred_element_type=jnp.float32)
        acc = acc + jnp.dot(u1.astype(jnp.bfloat16), w1_ref[...],
                            preferred_element_type=jnp.float32)
        acc = acc + jnp.dot(u2.astype(jnp.bfloat16), w2_ref[...],
                            preferred_element_type=jnp.float32)
        acc = acc + jnp.dot(hb, ws_ref[...],
                            preferred_element_type=jnp.float32)
        acc = acc + bias_ref[...]
        out_s[pl.ds(i * _S, _S), :] = acc
        if shortcut:
            h_s[pl.ds(i * _S, _S), :] = h
        s1_s[pl.ds(i, 1), :] = jnp.sum(acc, axis=0, keepdims=True)
        s2_s[pl.ds(i, 1), :] = jnp.sum(acc * acc, axis=0, keepdims=True)

    @pl.when(i == _B)
    def _bn1():
        inv_n = 1.0 / _N
        m1 = jnp.sum(s1_s[...], axis=0, keepdims=True) * inv_n
        v1 = jnp.sum(s2_s[...], axis=0, keepdims=True) * inv_n - m1 * m1
        sc1 = g1_ref[...] / jnp.sqrt(v1 + _EPS)
        sh1 = b1_ref[...] - m1 * sc1
        s1 = jnp.zeros((1, _H), jnp.float32)
        s2 = jnp.zeros((1, _H), jnp.float32)
        for c in range(_B):
            x = out_s[pl.ds(c * _S, _S), :]
            y = jnp.maximum(x * sc1 + sh1, 0.0)
            if shortcut:
                y = y + h_s[pl.ds(c * _S, _S), :]
            out_s[pl.ds(c * _S, _S), :] = y
            s1 = s1 + jnp.sum(y, axis=0, keepdims=True)
            s2 = s2 + jnp.sum(y * y, axis=0, keepdims=True)
        m2 = s1 * inv_n
        v2 = s2 * inv_n - m2 * m2
        sc2 = g2_ref[...] / jnp.sqrt(v2 + _EPS)
        st2_s[0:1, :] = sc2
        st2_s[1:2, :] = b2_ref[...] - m2 * sc2

    @pl.when(i > _B)
    def _bn2():
        b = i - _B - 1
        y = out_s[pl.ds(b * _S, _S), :]
        out_ref[...] = y * st2_s[0:1, :] + st2_s[1:2, :]


def kernel(n_coords, ca_coords, c_coords, Wl, bl, Ws, bs, g1, b1, g2, b2):
    ca = ca_coords.astype(jnp.float32)
    ca_cols = ca.transpose(0, 2, 1)
    adj = pl.pallas_call(
        _adj_kernel,
        grid=(_B, _S // _ROWS),
        in_specs=[pl.BlockSpec((1, _ROWS, 3), lambda b, r: (b, r, 0)),
                  pl.BlockSpec((1, 3, _S), lambda b, r: (b, 0, 0))],
        out_specs=pl.BlockSpec((1, _ROWS, _S), lambda b, r: (b, r, 0)),
        out_shape=jax.ShapeDtypeStruct((_B, _S, _S), jnp.bfloat16),
    )(ca, ca_cols)

    h = ca.reshape(_N, 3)
    for l in range(_NL):
        din = h.shape[1]
        shortcut = l > 0
        w = Wl[l]
        wb = [x.astype(jnp.bfloat16)
              for x in (w[:din], w[din:2 * din], w[2 * din:3 * din], Ws[l])]
        bias = (bl[l] + bs[l]).reshape(1, _H)
        blocked = pl.BlockSpec((_S, din), lambda i: (jnp.minimum(i, _B - 1), 0))
        const_w = pl.BlockSpec((din, _H), lambda i: (0, 0))
        const_v = pl.BlockSpec((1, _H), lambda i: (0, 0))
        scratch = [pltpu.VMEM((_N, _H), jnp.float32),
                   pltpu.VMEM((_N, din), jnp.float32) if shortcut else None,
                   pltpu.VMEM((_B, _H), jnp.float32),
                   pltpu.VMEM((_B, _H), jnp.float32),
                   pltpu.VMEM((2, _H), jnp.float32)]
        body = functools.partial(_layer_kernel, din=din, shortcut=shortcut)
        if not shortcut:
            scratch[1] = pltpu.VMEM((8, 8), jnp.float32)
        h = pl.pallas_call(
            body,
            grid=(2 * _B + 1,),
            in_specs=[blocked,
                      pl.BlockSpec((1, _S, _S),
                                   lambda i: (jnp.minimum(i, _B - 1), 0, 0)),
                      const_w, const_w, const_w, const_w,
                      const_v, const_v, const_v, const_v, const_v],
            out_specs=pl.BlockSpec(
                (_S, _H), lambda i: (jnp.maximum(i - _B - 1, 0), 0)),
            out_shape=jax.ShapeDtypeStruct((_N, _H), jnp.float32),
            scratch_shapes=scratch,
        )(h, adj, *wb, bias, g1[l].reshape(1, _H), b1[l].reshape(1, _H),
          g2[l].reshape(1, _H), b2[l].reshape(1, _H))
    return h.reshape(_B, _S, _H)


# speedup vs baseline: 1.0171x; 1.0071x over previous
import functools

import jax
import jax.numpy as jnp
from jax.experimental import pallas as pl
from jax.experimental.pallas import tpu as pltpu

_B, _S, _H = 8, 1024, 512
_N = _B * _S
_K = 10
_NL = 4
_EPS = 1e-5
_ROWS = 1024


def _adj_kernel(car_ref, cac_ref, adj_ref):
    rblk = pl.program_id(1)
    xr = car_ref[0]
    xc = cac_ref[0]
    d0 = xr[:, 0:1] - xc[0:1, :]
    d1 = xr[:, 1:2] - xc[1:2, :]
    d2 = xr[:, 2:3] - xc[2:3, :]
    dist = jnp.sqrt(d0 * d0 + d1 * d1 + d2 * d2)
    col = jax.lax.broadcasted_iota(jnp.int32, (_ROWS, _S), 1)
    row = rblk * _ROWS + jax.lax.broadcasted_iota(jnp.int32, (_ROWS, 1), 0)

    def body(_, neg):
        jstar = jnp.argmax(neg, axis=1).reshape(_ROWS, 1)
        return jnp.where(col == jstar, -jnp.inf, neg)

    neg0 = jnp.where(col == row, -jnp.inf, -dist)
    neg = jax.lax.fori_loop(0, _K, body, neg0)
    picked = (neg == -jnp.inf) & (col != row)
    adj_ref[0] = picked.astype(jnp.bfloat16)


def _layer_kernel(h_ref, a_ref, w0_ref, w1_ref, w2_ref, ws_ref, bias_ref,
                  g1_ref, b1_ref, g2_ref, b2_ref, out_ref,
                  out_s, h_s, s1_s, s2_s, st2_s, *, din, shortcut):
    i = pl.program_id(0)

    @pl.when(i < _B)
    def _mm():
        h = h_ref[...]
        a = a_ref[0]
        z = jnp.zeros((3, din), jnp.float32)
        hp = jnp.concatenate([z, h, z], axis=0)
        u0 = hp[4:4 + _S] + hp[5:5 + _S] + hp[6:6 + _S]
        u1 = hp[2:2 + _S] + hp[1:1 + _S] + hp[0:_S]
        hb = h.astype(jnp.bfloat16)
        u2 = jax.lax.dot_general(a, hb, (((0,), (0,)), ((), ())),
                                 preferred_element_type=jnp.float32)
        acc = jnp.dot(u0.astype(jnp.bfloat16), w0_ref[...],
                      preferred_element_type=jnp.float32)
        acc = acc + jnp.dot(u1.astype(jnp.bfloat16), w1_ref[...],
                            preferred_element_type=jnp.float32)
        acc = acc + jnp.dot(u2.astype(jnp.bfloat16), w2_ref[...],
                            preferred_element_type=jnp.float32)
        acc = acc + jnp.dot(hb, ws_ref[...],
                            preferred_element_type=jnp.float32)
        acc = acc + bias_ref[...]
        out_s[pl.ds(i * _S, _S), :] = acc
        if shortcut:
            h_s[pl.ds(i * _S, _S), :] = h
        s1_s[pl.ds(i, 1), :] = jnp.sum(acc, axis=0, keepdims=True)
        s2_s[pl.ds(i, 1), :] = jnp.sum(acc * acc, axis=0, keepdims=True)

    @pl.when(i == _B)
    def _bn1():
        inv_n = 1.0 / _N
        m1 = jnp.sum(s1_s[...], axis=0, keepdims=True) * inv_n
        v1 = jnp.sum(s2_s[...], axis=0, keepdims=True) * inv_n - m1 * m1
        sc1 = g1_ref[...] / jnp.sqrt(v1 + _EPS)
        sh1 = b1_ref[...] - m1 * sc1
        s1 = jnp.zeros((1, _H), jnp.float32)
        s2 = jnp.zeros((1, _H), jnp.float32)
        for c in range(_B):
            x = out_s[pl.ds(c * _S, _S), :]
            y = jnp.maximum(x * sc1 + sh1, 0.0)
            if shortcut:
                y = y + h_s[pl.ds(c * _S, _S), :]
            out_s[pl.ds(c * _S, _S), :] = y
            s1 = s1 + jnp.sum(y, axis=0, keepdims=True)
            s2 = s2 + jnp.sum(y * y, axis=0, keepdims=True)
        m2 = s1 * inv_n
        v2 = s2 * inv_n - m2 * m2
        sc2 = g2_ref[...] / jnp.sqrt(v2 + _EPS)
        st2_s[0:1, :] = sc2
        st2_s[1:2, :] = b2_ref[...] - m2 * sc2

    @pl.when(i > _B)
    def _bn2():
        b = i - _B - 1
        y = out_s[pl.ds(b * _S, _S), :]
        out_ref[...] = y * st2_s[0:1, :] + st2_s[1:2, :]


def kernel(n_coords, ca_coords, c_coords, Wl, bl, Ws, bs, g1, b1, g2, b2):
    ca = ca_coords.astype(jnp.float32)
    ca_cols = ca.transpose(0, 2, 1)
    adj = pl.pallas_call(
        _adj_kernel,
        grid=(_B, _S // _ROWS),
        in_specs=[pl.BlockSpec((1, _ROWS, 3), lambda b, r: (b, r, 0)),
                  pl.BlockSpec((1, 3, _S), lambda b, r: (b, 0, 0))],
        out_specs=pl.BlockSpec((1, _ROWS, _S), lambda b, r: (b, r, 0)),
        out_shape=jax.ShapeDtypeStruct((_B, _S, _S), jnp.bfloat16),
    )(ca, ca_cols)

    h = ca.reshape(_N, 3)
    for l in range(_NL):
        din = h.shape[1]
        shortcut = l > 0
        w = Wl[l]
        wb = [x.astype(jnp.bfloat16)
              for x in (w[:din], w[din:2 * din], w[2 * din:3 * din], Ws[l])]
        bias = (bl[l] + bs[l]).reshape(1, _H)
        blocked = pl.BlockSpec((_S, din), lambda i: (jnp.minimum(i, _B - 1), 0))
        const_w = pl.BlockSpec((din, _H), lambda i: (0, 0))
        const_v = pl.BlockSpec((1, _H), lambda i: (0, 0))
        scratch = [pltpu.VMEM((_N, _H), jnp.float32),
                   pltpu.VMEM((_N, din), jnp.float32) if shortcut else None,
                   pltpu.VMEM((_B, _H), jnp.float32),
                   pltpu.VMEM((_B, _H), jnp.float32),
                   pltpu.VMEM((2, _H), jnp.float32)]
        body = functools.partial(_layer_kernel, din=din, shortcut=shortcut)
        if not shortcut:
            scratch[1] = pltpu.VMEM((8, 8), jnp.float32)
        h = pl.pallas_call(
            body,
            grid=(2 * _B + 1,),
            in_specs=[blocked,
                      pl.BlockSpec((1, _S, _S),
                                   lambda i: (jnp.minimum(i, _B - 1), 0, 0)),
                      const_w, const_w, const_w, const_w,
                      const_v, const_v, const_v, const_v, const_v],
            out_specs=pl.BlockSpec(
                (_S, _H), lambda i: (jnp.maximum(i - _B - 1, 0), 0)),
            out_shape=jax.ShapeDtypeStruct((_N, _H), jnp.float32),
            scratch_shapes=scratch,
        )(h, adj, *wb, bias, g1[l].reshape(1, _H), b1[l].reshape(1, _H),
          g2[l].reshape(1, _H), b2[l].reshape(1, _H))
    return h.reshape(_B, _S, _H)
